# baseline (speedup 1.0000x reference)
import functools

import jax
import jax.numpy as jnp
from jax import lax
from jax.experimental import pallas as pl
from jax.experimental.pallas import tpu as pltpu
from jax.experimental.pallas import tpu_sc as plsc

B, L, D = 4, 4096, 2048
E, R, K = 8, 8, 2
EP = 16
ALPHA = 1.0 / R
NEG = -1e30
LCHUNK = 512
NL = L // LCHUNK
LCHUNK_A = 2048
NLA = L // LCHUNK_A
NC16 = D // 16


def _mean_logits_kernel(x_ref, gw_ref, gb_ref, hsum_ref, logits_ref):
    li = pl.program_id(1)

    @pl.when(li == 0)
    def _():
        hsum_ref[...] = jnp.zeros_like(hsum_ref)

    hsum_ref[0] += jnp.sum(x_ref[0], axis=0, keepdims=True)

    @pl.when(li == NLA - 1)
    def _():
        h = hsum_ref[0] * (1.0 / L)
        hsum_ref[0] = h
        lg8 = (jnp.dot(h, gw_ref[...].T, preferred_element_type=jnp.float32)
               + gb_ref[...])
        logits_ref[0] = jnp.concatenate(
            [lg8, jnp.full((1, EP - E), NEG, jnp.float32)], axis=1)


def _vsum(v, n=16):
    parts = [v[i] for i in range(n)]
    while len(parts) > 1:
        parts = [a + b for a, b in zip(parts[::2], parts[1::2])]
    return parts[0]


def _recip(s, iota):
    return (1.0 / jnp.where(iota == 0, s, 1.0))[0]


def _top2(v):
    l = [v[i] for i in range(E)]
    m1, i1 = l[0], jnp.int32(0)
    for i in range(1, E):
        gt = l[i] > m1
        i1 = jnp.where(gt, jnp.int32(i), i1)
        m1 = jnp.where(gt, l[i], m1)
    l2 = [jnp.where(jnp.int32(i) == i1, NEG, l[i]) for i in range(E)]
    m2, i2 = l2[0], jnp.int32(0)
    for i in range(1, E):
        gt = l2[i] > m2
        i2 = jnp.where(gt, jnp.int32(i), i2)
        m2 = jnp.where(gt, l2[i], m2)
    return m1, i1, m2, i2


def _sc_row_worker(jb, h_hbm, lg_hbm, a_hbm, b_hbm, delta_hbm,
                   lgrow_v, h_v, a_v, b_v, out_v, sem, semb, semh):
    ch = pltpu.async_copy(h_hbm.at[jb, 0], h_v, semh)
    pltpu.sync_copy(lg_hbm.at[jb, 0], lgrow_v)
    iota = lax.iota(jnp.int32, 16)
    v = lgrow_v[...]
    m1, i1, m2, i2 = _top2(v)
    e21 = jnp.exp(jnp.where(iota == 0, m2 - m1, 0.0))[0]
    w1 = _recip(1.0 + e21, iota)
    w2 = 1.0 - w1
    half = iota >= R
    evec = jnp.where(half, i2, i1)
    idx = evec * R + (iota - jnp.where(half, R, 0))
    ca = pltpu.async_copy(a_hbm.at[idx], a_v, sem)
    cb = pltpu.async_copy(b_hbm.at[idx], b_v, semb)
    ch.wait()
    ca.wait()

    def zbody(c, accs):
        s = pl.ds(c * 16, 16)
        hv = h_v[s]
        return tuple(accs[p] + a_v[p, s] * hv for p in range(2 * R))

    accs = lax.fori_loop(0, NC16, zbody,
                         tuple(jnp.zeros((16,), jnp.float32)
                               for _ in range(2 * R)))
    zcs = [_vsum(accs[p]) * (w2 if p >= R else w1) * ALPHA
           for p in range(2 * R)]
    cb.wait()

    def dbody(c, carry):
        for j in range(4):
            s = pl.ds(c * 64 + j * 16, 16)
            acc = zcs[0] * b_v[0, s]
            for p in range(1, 2 * R):
                acc += zcs[p] * b_v[p, s]
            out_v[s] = acc
        return carry

    lax.fori_loop(0, NC16 // 4, dbody, 0)
    pltpu.sync_copy(out_v, delta_hbm.at[jb, 0])


def _sc_aux_worker(lg_rows, pieces_hbm, lgrow_v, pc_v):
    iota = lax.iota(jnp.int32, 16)
    counts = jnp.zeros((16,), jnp.float32)
    Pacc = jnp.zeros((16,), jnp.float32)
    pieces = jnp.zeros((16,), jnp.float32)
    for b, (ref, row) in enumerate(lg_rows):
        pltpu.sync_copy(ref.at[row, 0], lgrow_v)
        v = lgrow_v[...]
        m1, i1, m2, i2 = _top2(v)
        ex = jnp.exp(v - m1)
        S = _vsum(ex)
        Pacc = Pacc + ex * _recip(S, iota)
        counts = (counts
                  + jnp.where(iota == i1, 1.0, 0.0)
                  + jnp.where(iota == i2, 1.0, 0.0))
        pieces = jnp.where(iota == 2 + b, S, pieces)
        pieces = jnp.where(iota == 6 + b, m1, pieces)
    Pmean = Pacc * (1.0 / B)
    f = counts * (1.0 / (B * K))
    lb = E * _vsum(f * Pmean)
    p = jnp.where(iota < E, f + 1e-8, 0.0)
    p = p * _recip(_vsum(p), iota)
    sum_p2 = _vsum(p * p)
    pieces = jnp.where(iota == 0, lb, pieces)
    pieces = jnp.where(iota == 1, sum_p2, pieces)
    pc_v[...] = pieces
    pltpu.sync_copy(pc_v, pieces_hbm.at[0])


_SC_MESH = plsc.VectorSubcoreMesh(core_axis_name="c", subcore_axis_name="s")
_SC_SCRATCH = [
    pltpu.VMEM((16,), jnp.float32),
    pltpu.VMEM((D,), jnp.float32),
    pltpu.VMEM((2 * R, D), jnp.float32),
    pltpu.VMEM((2 * R, D), jnp.float32),
    pltpu.VMEM((D,), jnp.float32),
    pltpu.VMEM((16,), jnp.float32),
    pltpu.SemaphoreType.DMA,
    pltpu.SemaphoreType.DMA,
    pltpu.SemaphoreType.DMA,
]


@functools.partial(
    pl.kernel, mesh=_SC_MESH,
    out_type=[jax.ShapeDtypeStruct((2, 1, D), jnp.float32)],
    scratch_types=_SC_SCRATCH,
)
def _sc_route01(h_hbm, lg_hbm, a_hbm, b_hbm, delta_hbm,
                lgrow_v, h_v, a_v, b_v, out_v, pc_v, sem, semb, semh):
    cid = lax.axis_index("c")
    sid = lax.axis_index("s")
    for jb in range(2):
        @pl.when((cid == jb) & (sid == 0))
        def _(jb=jb):
            _sc_row_worker(jb, h_hbm, lg_hbm, a_hbm, b_hbm, delta_hbm,
                           lgrow_v, h_v, a_v, b_v, out_v, sem, semb, semh)


@functools.partial(
    pl.kernel, mesh=_SC_MESH,
    out_type=[
        jax.ShapeDtypeStruct((2, 1, D), jnp.float32),
        jax.ShapeDtypeStruct((1, 16), jnp.float32),
    ],
    scratch_types=_SC_SCRATCH,
)
def _sc_route23(h_hbm, lg01_hbm, lg_hbm, a_hbm, b_hbm, delta_hbm, pieces_hbm,
                lgrow_v, h_v, a_v, b_v, out_v, pc_v, sem, semb, semh):
    cid = lax.axis_index("c")
    sid = lax.axis_index("s")
    for jb in range(2):
        @pl.when((cid == jb) & (sid == 0))
        def _(jb=jb):
            _sc_row_worker(jb, h_hbm, lg_hbm, a_hbm, b_hbm, delta_hbm,
                           lgrow_v, h_v, a_v, b_v, out_v, sem, semb, semh)

    @pl.when((cid == 0) & (sid == 1))
    def _():
        _sc_aux_worker(
            [(lg01_hbm, 0), (lg01_hbm, 1), (lg_hbm, 0), (lg_hbm, 1)],
            pieces_hbm, lgrow_v, pc_v)


def _ln_kernel(x_ref, delta_ref, g_ref, bta_ref, o_ref):
    y = x_ref[0] + delta_ref[0]
    mu = jnp.mean(y, axis=1, keepdims=True)
    yc = y - mu
    var = jnp.mean(yc * yc, axis=1, keepdims=True)
    o_ref[0] = yc * jax.lax.rsqrt(var + 1e-5) * g_ref[...] + bta_ref[...]


def _ln_aux_kernel(prev_ref, x_ref, delta_ref, g_ref, bta_ref, pc_ref,
                   o_ref, aux_ref):
    del prev_ref
    bi = pl.program_id(0)
    li = pl.program_id(1)

    @pl.when((bi == 0) & (li == 0))
    def _():
        pc = pc_ref[0]
        log_z = jnp.log(pc[2:6]) + pc[6:10]
        z_loss = jnp.mean(log_z ** 2)
        aux = 0.01 * pc[0] + 0.001 * z_loss + 0.01 * jnp.log(pc[1])
        aux_ref[...] = jnp.full_like(aux_ref, aux)

    y = x_ref[0] + delta_ref[0]
    mu = jnp.mean(y, axis=1, keepdims=True)
    yc = y - mu
    var = jnp.mean(yc * yc, axis=1, keepdims=True)
    o_ref[0] = yc * jax.lax.rsqrt(var + 1e-5) * g_ref[...] + bta_ref[...]


def _mean_logits(x, gate_w, gb2, off):
    return pl.pallas_call(
        _mean_logits_kernel,
        grid=(2, NLA),
        in_specs=[
            pl.BlockSpec((1, LCHUNK_A, D), lambda b, l: (b + off, l, 0)),
            pl.BlockSpec((E, D), lambda b, l: (0, 0)),
            pl.BlockSpec((1, E), lambda b, l: (0, 0)),
        ],
        out_specs=[
            pl.BlockSpec((1, 1, D), lambda b, l: (b, 0, 0)),
            pl.BlockSpec((1, 1, EP), lambda b, l: (b, 0, 0)),
        ],
        out_shape=[
            jax.ShapeDtypeStruct((2, 1, D), jnp.float32),
            jax.ShapeDtypeStruct((2, 1, EP), jnp.float32),
        ],
    )(x, gate_w, gb2)


@jax.jit
def kernel(x, gate_w, gate_b, A_stack, B_stack, ln_gamma, ln_beta):
    gb2 = gate_b.reshape(1, E)
    g2 = ln_gamma.reshape(1, D)
    bt2 = ln_beta.reshape(1, D)
    a_flat = A_stack.reshape(E * R, D)
    b_flat = B_stack.transpose(0, 2, 1).reshape(E * R, D)

    h01, lg01 = _mean_logits(x, gate_w, gb2, 0)
    h23, lg23 = _mean_logits(x, gate_w, gb2, 2)

    (delta01,) = _sc_route01(h01, lg01, a_flat, b_flat)
    delta23, pieces = _sc_route23(h23, lg01, lg23, a_flat, b_flat)

    out01 = pl.pallas_call(
        _ln_kernel,
        grid=(2, NL),
        in_specs=[
            pl.BlockSpec((1, LCHUNK, D), lambda b, l: (b, l, 0)),
            pl.BlockSpec((1, 1, D), lambda b, l: (b, 0, 0)),
            pl.BlockSpec((1, D), lambda b, l: (0, 0)),
            pl.BlockSpec((1, D), lambda b, l: (0, 0)),
        ],
        out_specs=pl.BlockSpec((1, LCHUNK, D), lambda b, l: (b, l, 0)),
        out_shape=jax.ShapeDtypeStruct((B, L, D), jnp.float32),
    )(x, delta01, g2, bt2)

    out, aux = pl.pallas_call(
        _ln_aux_kernel,
        grid=(2, NL),
        in_specs=[
            pl.BlockSpec(memory_space=pl.MemorySpace.ANY),
            pl.BlockSpec((1, LCHUNK, D), lambda b, l: (b + 2, l, 0)),
            pl.BlockSpec((1, 1, D), lambda b, l: (b, 0, 0)),
            pl.BlockSpec((1, D), lambda b, l: (0, 0)),
            pl.BlockSpec((1, D), lambda b, l: (0, 0)),
            pl.BlockSpec((1, 16), lambda b, l: (0, 0)),
        ],
        out_specs=[
            pl.BlockSpec((1, LCHUNK, D), lambda b, l: (b + 2, l, 0)),
            pl.BlockSpec((8, 128), lambda b, l: (0, 0)),
        ],
        out_shape=[
            jax.ShapeDtypeStruct((B, L, D), jnp.float32),
            jax.ShapeDtypeStruct((8, 128), jnp.float32),
        ],
        input_output_aliases={0: 0},
    )(out01, x, delta23, g2, bt2, pieces)

    return out, aux[0, 0]


# speedup vs baseline: 1.0316x; 1.0316x over previous
import functools

import jax
import jax.numpy as jnp
from jax import lax
from jax.experimental import pallas as pl
from jax.experimental.pallas import tpu as pltpu
from jax.experimental.pallas import tpu_sc as plsc

B, L, D = 4, 4096, 2048
E, R, K = 8, 8, 2
EP = 16
ALPHA = 1.0 / R
NEG = -1e30
LCHUNK = 1024
NL = L // LCHUNK
LCHUNK_A = 1024
NLA = L // LCHUNK_A
NC16 = D // 16


def _mean_logits_kernel(x_ref, gw_ref, gb_ref, hsum_ref, logits_ref):
    li = pl.program_id(1)

    @pl.when(li == 0)
    def _():
        hsum_ref[...] = jnp.zeros_like(hsum_ref)

    hsum_ref[0] += jnp.sum(x_ref[0], axis=0, keepdims=True)

    @pl.when(li == NLA - 1)
    def _():
        h = hsum_ref[0] * (1.0 / L)
        hsum_ref[0] = h
        lg8 = (jnp.dot(h, gw_ref[...].T, preferred_element_type=jnp.float32)
               + gb_ref[...])
        logits_ref[0] = jnp.concatenate(
            [lg8, jnp.full((1, EP - E), NEG, jnp.float32)], axis=1)


def _vsum(v, n=16):
    parts = [v[i] for i in range(n)]
    while len(parts) > 1:
        parts = [a + b for a, b in zip(parts[::2], parts[1::2])]
    return parts[0]


def _recip(s, iota):
    return (1.0 / jnp.where(iota == 0, s, 1.0))[0]


def _top2(v):
    l = [v[i] for i in range(E)]
    m1, i1 = l[0], jnp.int32(0)
    for i in range(1, E):
        gt = l[i] > m1
        i1 = jnp.where(gt, jnp.int32(i), i1)
        m1 = jnp.where(gt, l[i], m1)
    l2 = [jnp.where(jnp.int32(i) == i1, NEG, l[i]) for i in range(E)]
    m2, i2 = l2[0], jnp.int32(0)
    for i in range(1, E):
        gt = l2[i] > m2
        i2 = jnp.where(gt, jnp.int32(i), i2)
        m2 = jnp.where(gt, l2[i], m2)
    return m1, i1, m2, i2


def _sc_row_worker(jb, h_hbm, lg_hbm, a_hbm, b_hbm, delta_hbm,
                   lgrow_v, h_v, a_v, b_v, out_v, sem, semb, semh):
    ch = pltpu.async_copy(h_hbm.at[jb, 0], h_v, semh)
    pltpu.sync_copy(lg_hbm.at[jb, 0], lgrow_v)
    iota = lax.iota(jnp.int32, 16)
    v = lgrow_v[...]
    m1, i1, m2, i2 = _top2(v)
    e21 = jnp.exp(jnp.where(iota == 0, m2 - m1, 0.0))[0]
    w1 = _recip(1.0 + e21, iota)
    w2 = 1.0 - w1
    half = iota >= R
    evec = jnp.where(half, i2, i1)
    idx = evec * R + (iota - jnp.where(half, R, 0))
    ca = pltpu.async_copy(a_hbm.at[idx], a_v, sem)
    cb = pltpu.async_copy(b_hbm.at[idx], b_v, semb)
    ch.wait()
    ca.wait()

    def zbody(c, accs):
        s = pl.ds(c * 16, 16)
        hv = h_v[s]
        return tuple(accs[p] + a_v[p, s] * hv for p in range(2 * R))

    accs = lax.fori_loop(0, NC16, zbody,
                         tuple(jnp.zeros((16,), jnp.float32)
                               for _ in range(2 * R)))
    zcs = [_vsum(accs[p]) * (w2 if p >= R else w1) * ALPHA
           for p in range(2 * R)]
    cb.wait()

    def dbody(c, carry):
        for j in range(4):
            s = pl.ds(c * 64 + j * 16, 16)
            acc = zcs[0] * b_v[0, s]
            for p in range(1, 2 * R):
                acc += zcs[p] * b_v[p, s]
            out_v[s] = acc
        return carry

    lax.fori_loop(0, NC16 // 4, dbody, 0)
    pltpu.sync_copy(out_v, delta_hbm.at[jb, 0])


def _sc_aux_worker(lg_rows, pieces_hbm, lgrow_v, pc_v):
    iota = lax.iota(jnp.int32, 16)
    counts = jnp.zeros((16,), jnp.float32)
    Pacc = jnp.zeros((16,), jnp.float32)
    pieces = jnp.zeros((16,), jnp.float32)
    for b, (ref, row) in enumerate(lg_rows):
        pltpu.sync_copy(ref.at[row, 0], lgrow_v)
        v = lgrow_v[...]
        m1, i1, m2, i2 = _top2(v)
        ex = jnp.exp(v - m1)
        S = _vsum(ex)
        Pacc = Pacc + ex * _recip(S, iota)
        counts = (counts
                  + jnp.where(iota == i1, 1.0, 0.0)
                  + jnp.where(iota == i2, 1.0, 0.0))
        pieces = jnp.where(iota == 2 + b, S, pieces)
        pieces = jnp.where(iota == 6 + b, m1, pieces)
    Pmean = Pacc * (1.0 / B)
    f = counts * (1.0 / (B * K))
    lb = E * _vsum(f * Pmean)
    p = jnp.where(iota < E, f + 1e-8, 0.0)
    p = p * _recip(_vsum(p), iota)
    sum_p2 = _vsum(p * p)
    pieces = jnp.where(iota == 0, lb, pieces)
    pieces = jnp.where(iota == 1, sum_p2, pieces)
    pc_v[...] = pieces
    pltpu.sync_copy(pc_v, pieces_hbm.at[0])


_SC_MESH = plsc.VectorSubcoreMesh(core_axis_name="c", subcore_axis_name="s")
_SC_SCRATCH = [
    pltpu.VMEM((16,), jnp.float32),
    pltpu.VMEM((D,), jnp.float32),
    pltpu.VMEM((2 * R, D), jnp.float32),
    pltpu.VMEM((2 * R, D), jnp.float32),
    pltpu.VMEM((D,), jnp.float32),
    pltpu.VMEM((16,), jnp.float32),
    pltpu.SemaphoreType.DMA,
    pltpu.SemaphoreType.DMA,
    pltpu.SemaphoreType.DMA,
]


@functools.partial(
    pl.kernel, mesh=_SC_MESH,
    out_type=[jax.ShapeDtypeStruct((2, 1, D), jnp.float32)],
    scratch_types=_SC_SCRATCH,
)
def _sc_route01(h_hbm, lg_hbm, a_hbm, b_hbm, delta_hbm,
                lgrow_v, h_v, a_v, b_v, out_v, pc_v, sem, semb, semh):
    cid = lax.axis_index("c")
    sid = lax.axis_index("s")
    for jb in range(2):
        @pl.when((cid == jb) & (sid == 0))
        def _(jb=jb):
            _sc_row_worker(jb, h_hbm, lg_hbm, a_hbm, b_hbm, delta_hbm,
                           lgrow_v, h_v, a_v, b_v, out_v, sem, semb, semh)


@functools.partial(
    pl.kernel, mesh=_SC_MESH,
    out_type=[
        jax.ShapeDtypeStruct((2, 1, D), jnp.float32),
        jax.ShapeDtypeStruct((1, 16), jnp.float32),
    ],
    scratch_types=_SC_SCRATCH,
)
def _sc_route23(h_hbm, lg01_hbm, lg_hbm, a_hbm, b_hbm, delta_hbm, pieces_hbm,
                lgrow_v, h_v, a_v, b_v, out_v, pc_v, sem, semb, semh):
    cid = lax.axis_index("c")
    sid = lax.axis_index("s")
    for jb in range(2):
        @pl.when((cid == jb) & (sid == 0))
        def _(jb=jb):
            _sc_row_worker(jb, h_hbm, lg_hbm, a_hbm, b_hbm, delta_hbm,
                           lgrow_v, h_v, a_v, b_v, out_v, sem, semb, semh)

    @pl.when((cid == 0) & (sid == 1))
    def _():
        _sc_aux_worker(
            [(lg01_hbm, 0), (lg01_hbm, 1), (lg_hbm, 0), (lg_hbm, 1)],
            pieces_hbm, lgrow_v, pc_v)


def _ln_kernel(x_ref, delta_ref, g_ref, bta_ref, o_ref):
    y = x_ref[0] + delta_ref[0]
    mu = jnp.mean(y, axis=1, keepdims=True)
    yc = y - mu
    var = jnp.mean(yc * yc, axis=1, keepdims=True)
    o_ref[0] = yc * jax.lax.rsqrt(var + 1e-5) * g_ref[...] + bta_ref[...]


def _ln_aux_kernel(prev_ref, x_ref, delta_ref, g_ref, bta_ref, pc_ref,
                   o_ref, aux_ref):
    del prev_ref
    bi = pl.program_id(0)
    li = pl.program_id(1)

    @pl.when((bi == 0) & (li == 0))
    def _():
        pc = pc_ref[0]
        log_z = jnp.log(pc[2:6]) + pc[6:10]
        z_loss = jnp.mean(log_z ** 2)
        aux = 0.01 * pc[0] + 0.001 * z_loss + 0.01 * jnp.log(pc[1])
        aux_ref[...] = jnp.full_like(aux_ref, aux)

    y = x_ref[0] + delta_ref[0]
    mu = jnp.mean(y, axis=1, keepdims=True)
    yc = y - mu
    var = jnp.mean(yc * yc, axis=1, keepdims=True)
    o_ref[0] = yc * jax.lax.rsqrt(var + 1e-5) * g_ref[...] + bta_ref[...]


def _mean_logits(x, gate_w, gb2, off):
    return pl.pallas_call(
        _mean_logits_kernel,
        grid=(2, NLA),
        in_specs=[
            pl.BlockSpec((1, LCHUNK_A, D), lambda b, l: (b + off, l, 0)),
            pl.BlockSpec((E, D), lambda b, l: (0, 0)),
            pl.BlockSpec((1, E), lambda b, l: (0, 0)),
        ],
        out_specs=[
            pl.BlockSpec((1, 1, D), lambda b, l: (b, 0, 0)),
            pl.BlockSpec((1, 1, EP), lambda b, l: (b, 0, 0)),
        ],
        out_shape=[
            jax.ShapeDtypeStruct((2, 1, D), jnp.float32),
            jax.ShapeDtypeStruct((2, 1, EP), jnp.float32),
        ],
    )(x, gate_w, gb2)


@jax.jit
def kernel(x, gate_w, gate_b, A_stack, B_stack, ln_gamma, ln_beta):
    gb2 = gate_b.reshape(1, E)
    g2 = ln_gamma.reshape(1, D)
    bt2 = ln_beta.reshape(1, D)
    a_flat = A_stack.reshape(E * R, D)
    b_flat = B_stack.transpose(0, 2, 1).reshape(E * R, D)

    h01, lg01 = _mean_logits(x, gate_w, gb2, 0)
    h23, lg23 = _mean_logits(x, gate_w, gb2, 2)

    (delta01,) = _sc_route01(h01, lg01, a_flat, b_flat)
    delta23, pieces = _sc_route23(h23, lg01, lg23, a_flat, b_flat)

    out01 = pl.pallas_call(
        _ln_kernel,
        grid=(2, NL),
        in_specs=[
            pl.BlockSpec((1, LCHUNK, D), lambda b, l: (b, l, 0)),
            pl.BlockSpec((1, 1, D), lambda b, l: (b, 0, 0)),
            pl.BlockSpec((1, D), lambda b, l: (0, 0)),
            pl.BlockSpec((1, D), lambda b, l: (0, 0)),
        ],
        out_specs=pl.BlockSpec((1, LCHUNK, D), lambda b, l: (b, l, 0)),
        out_shape=jax.ShapeDtypeStruct((B, L, D), jnp.float32),
    )(x, delta01, g2, bt2)

    out, aux = pl.pallas_call(
        _ln_aux_kernel,
        grid=(2, NL),
        in_specs=[
            pl.BlockSpec(memory_space=pl.MemorySpace.ANY),
            pl.BlockSpec((1, LCHUNK, D), lambda b, l: (b + 2, l, 0)),
            pl.BlockSpec((1, 1, D), lambda b, l: (b, 0, 0)),
            pl.BlockSpec((1, D), lambda b, l: (0, 0)),
            pl.BlockSpec((1, D), lambda b, l: (0, 0)),
            pl.BlockSpec((1, 16), lambda b, l: (0, 0)),
        ],
        out_specs=[
            pl.BlockSpec((1, LCHUNK, D), lambda b, l: (b + 2, l, 0)),
            pl.BlockSpec((8, 128), lambda b, l: (0, 0)),
        ],
        out_shape=[
            jax.ShapeDtypeStruct((B, L, D), jnp.float32),
            jax.ShapeDtypeStruct((8, 128), jnp.float32),
        ],
        input_output_aliases={0: 0},
    )(out01, x, delta23, g2, bt2, pieces)

    return out, aux[0, 0]
